# baseline (speedup 1.0000x reference)
import functools

import jax
import jax.numpy as jnp
from jax import lax
from jax.experimental import pallas as pl
from jax.experimental.pallas import tpu as pltpu
from jax.experimental.pallas import tpu_sc as plsc

B, H, D = 32, 32, 128
KV = 8192
K = 2048
NLANE = 16
CHUNK = KV // NLANE
DIGIT_BITS = 7
RADIX = 1 << DIGIT_BITS
NPASS = 5
NB = 8
SUB = CHUNK // NB
NBINS = RADIX * NLANE

T_BLK = 2048
NST = 4


def _scores_body(q_ref, w_ref, *refs):
    kv_refs = refs[:NST]
    scr_ref, out_ref = refs[NST], refs[NST + 1]
    b = pl.program_id(0)
    t = pl.program_id(1)
    q = q_ref[0]
    w = w_ref[b][None, :]
    scale = jnp.float32(D) ** -0.5
    for j in range(NST):
        kv = kv_refs[j][0]
        logits = lax.dot_general(
            q, kv, (((1,), (1,)), ((), ())),
            preferred_element_type=jnp.float32)
        act = jnp.maximum(logits * scale, 0.0)
        s = lax.dot_general(
            w, act, (((1,), (0,)), ((), ())),
            preferred_element_type=jnp.float32)
        sl = pl.ds((t * NST + j) * T_BLK, T_BLK)
        out_ref[0, 0, sl] = s[0] + scr_ref[0, 0, sl]


def _scores(query, weights, index_kv_cache, index_scratch):
    grid = (B, KV // (NST * T_BLK))

    def kv_spec(j):
        return pl.BlockSpec((1, T_BLK, D),
                            lambda b, t, j=j: (b, t * NST + j, 0))

    out = pl.pallas_call(
        _scores_body,
        grid=grid,
        in_specs=[
            pl.BlockSpec((1, H, D), lambda b, t: (b, 0, 0)),
            pl.BlockSpec((B, H), lambda b, t: (0, 0)),
        ] + [kv_spec(j) for j in range(NST)] + [
            pl.BlockSpec((1, 1, KV), lambda b, t: (b, 0, 0)),
        ],
        out_specs=pl.BlockSpec((1, 1, KV), lambda b, t: (b, 0, 0)),
        out_shape=jax.ShapeDtypeStruct((B, 1, KV), jnp.float32),
    )(query, weights,
      *[index_kv_cache] * NST,
      index_scratch.reshape(B, 1, KV))
    return out.reshape(B, KV)


def _topk_body(scores_hbm, lens_hbm, out_s_hbm, out_i_hbm,
               sc_v, key_a, key_b, idx_a, idx_b, lens_v,
               outs_v, outi_v, tots, bases, *hists):
    wid = lax.axis_index("s") * 2 + lax.axis_index("c")
    pltpu.sync_copy(scores_hbm.at[wid], sc_v)
    pltpu.sync_copy(lens_hbm.at[wid], lens_v)

    iota = lax.iota(jnp.int32, NLANE)
    len_vec = jnp.maximum(lens_v[...], jnp.int32(K))

    def pad(x):
        return x + lax.shift_right_logical(x, 9)

    @pl.loop(0, CHUNK, unroll=4)
    def _build(it):
        sl = pl.ds(it * NLANE, NLANE)
        bits = lax.bitcast_convert_type(sc_v[sl], jnp.int32)
        key_m = jnp.where(bits >= 0, bits ^ jnp.int32(-2**31), ~bits)
        invkey = ~key_m
        e = it * NLANE + iota
        invkey = jnp.where(e < len_vec, invkey, jnp.int32(-1))
        pa = pad(e)
        plsc.store_scatter(key_a, [pa], invkey)
        plsc.store_scatter(idx_a, [pa], e)

    ones = jnp.ones((NLANE,), jnp.int32)
    lane_base = iota * (CHUNK + 1)
    dmask = jnp.int32(RADIX - 1)

    for p in range(NPASS):
        shift = jnp.int32(DIGIT_BITS * p)
        src_k, src_i = (key_a, idx_a) if p % 2 == 0 else (key_b, idx_b)
        dst_k, dst_i = (key_b, idx_b) if p % 2 == 0 else (key_a, idx_a)

        @pl.loop(0, NBINS // NLANE, unroll=8)
        def _zero(i):
            sl = pl.ds(i * NLANE, NLANE)
            for hb in hists:
                hb[sl] = jnp.zeros((NLANE,), jnp.int32)

        @pl.loop(0, SUB)
        def _hist(it):
            for s, hb in enumerate(hists):
                k = plsc.load_gather(src_k, [lane_base + (s * SUB + it)])
                d = lax.shift_right_logical(k, shift) & dmask
                plsc.addupdate_scatter(hb, [d * NLANE + iota], ones)

        @pl.loop(0, RADIX, unroll=2)
        def _local_prefix(dd):
            sl = pl.ds(dd * NLANE, NLANE)
            c = [hb[sl] for hb in hists]
            tot = c[0]
            for s in range(1, NB):
                tot = tot + c[s]
            incl = plsc.cumsum(tot)
            run = incl - tot
            plsc.store_scatter(tots, [jnp.full((NLANE,), dd, jnp.int32)],
                               incl, mask=iota == jnp.int32(NLANE - 1))
            for s, hb in enumerate(hists):
                hb[sl] = run
                if s + 1 < NB:
                    run = run + c[s]

        def _digit_prefix(g, carry):
            sl = pl.ds(g * NLANE, NLANE)
            tv = tots[sl]
            ex = plsc.cumsum(tv) - tv
            bases[sl] = ex + carry
            return carry + jnp.sum(tv)

        lax.fori_loop(0, RADIX // NLANE, _digit_prefix, jnp.int32(0))

        @pl.loop(0, SUB, unroll=2)
        def _permute(it):
            for s, hb in enumerate(hists):
                a = lane_base + (s * SUB + it)
                k = plsc.load_gather(src_k, [a])
                v = plsc.load_gather(src_i, [a])
                d = lax.shift_right_logical(k, shift) & dmask
                cidx = d * NLANE + iota
                pos = plsc.load_gather(hb, [cidx]) + plsc.load_gather(bases, [d])
                pa = pad(pos)
                plsc.store_scatter(dst_k, [pa], k)
                plsc.store_scatter(dst_i, [pa], v)
                plsc.addupdate_scatter(hb, [cidx], ones)

    src_i_fin = idx_a if NPASS % 2 == 0 else idx_b

    @pl.loop(0, K // NLANE, unroll=4)
    def _emit(it):
        sl = pl.ds(it * NLANE, NLANE)
        rank = it * NLANE + iota
        i_vec = plsc.load_gather(src_i_fin, [pad(rank)])
        outs_v[sl] = plsc.load_gather(sc_v, [i_vec])
        outi_v[sl] = i_vec

    pltpu.sync_copy(outs_v, out_s_hbm.at[wid])
    pltpu.sync_copy(outi_v, out_i_hbm.at[wid])


def _topk(scores, kv_lens):
    mesh = plsc.VectorSubcoreMesh(core_axis_name="c", subcore_axis_name="s")
    fn = pl.kernel(
        _topk_body,
        out_type=(jax.ShapeDtypeStruct((B, K), jnp.float32),
                  jax.ShapeDtypeStruct((B, K), jnp.int32)),
        mesh=mesh,
        scratch_types=[
            pltpu.VMEM((KV,), jnp.float32),
            pltpu.VMEM((KV + NLANE,), jnp.int32),
            pltpu.VMEM((KV + NLANE,), jnp.int32),
            pltpu.VMEM((KV + NLANE,), jnp.int32),
            pltpu.VMEM((KV + NLANE,), jnp.int32),
            pltpu.VMEM((NLANE,), jnp.int32),
            pltpu.VMEM((K,), jnp.float32),
            pltpu.VMEM((K,), jnp.int32),
            pltpu.VMEM((RADIX,), jnp.int32),
            pltpu.VMEM((RADIX,), jnp.int32),
        ] + [pltpu.VMEM((NBINS,), jnp.int32) for _ in range(NB)],
        compiler_params=pltpu.CompilerParams(needs_layout_passes=False),
    )
    return fn(scores, kv_lens)


def kernel(query, weights, index_kv_cache, kv_lens, block_size, layer_id,
           index_scratch):
    scores = _scores(query, weights, index_kv_cache, index_scratch)
    lens_b = jnp.broadcast_to(
        kv_lens.astype(jnp.int32)[:, None], (B, NLANE))
    return _topk(scores, lens_b)


# speedup vs baseline: 1.3892x; 1.3892x over previous
import functools

import jax
import jax.numpy as jnp
from jax import lax
from jax.experimental import pallas as pl
from jax.experimental.pallas import tpu as pltpu
from jax.experimental.pallas import tpu_sc as plsc

B, H, D = 32, 32, 128
KV = 8192
K = 2048
NLANE = 16
CHUNK = KV // NLANE
DIGIT_BITS = 7
RADIX = 1 << DIGIT_BITS
NPASS = 5
NB = 8
SUB = CHUNK // NB
NBINS = RADIX * NLANE

T_BLK = 2048
NST = 4


def _scores_body(q_ref, w_ref, *refs):
    kv_refs = refs[:NST]
    scr_ref, out_ref = refs[NST], refs[NST + 1]
    b = pl.program_id(0)
    t = pl.program_id(1)
    q = q_ref[0]
    w = w_ref[b][None, :]
    scale = jnp.float32(D) ** -0.5
    for j in range(NST):
        kv = kv_refs[j][0]
        logits = lax.dot_general(
            q, kv, (((1,), (1,)), ((), ())),
            preferred_element_type=jnp.float32)
        act = jnp.maximum(logits * scale, 0.0)
        s = lax.dot_general(
            w, act, (((1,), (0,)), ((), ())),
            preferred_element_type=jnp.float32)
        sl = pl.ds((t * NST + j) * T_BLK, T_BLK)
        out_ref[0, 0, sl] = s[0] + scr_ref[0, 0, sl]


def _scores(query, weights, index_kv_cache, index_scratch):
    grid = (B, KV // (NST * T_BLK))

    def kv_spec(j):
        return pl.BlockSpec((1, T_BLK, D),
                            lambda b, t, j=j: (b, t * NST + j, 0))

    out = pl.pallas_call(
        _scores_body,
        grid=grid,
        in_specs=[
            pl.BlockSpec((1, H, D), lambda b, t: (b, 0, 0)),
            pl.BlockSpec((B, H), lambda b, t: (0, 0)),
        ] + [kv_spec(j) for j in range(NST)] + [
            pl.BlockSpec((1, 1, KV), lambda b, t: (b, 0, 0)),
        ],
        out_specs=pl.BlockSpec((1, 1, KV), lambda b, t: (b, 0, 0)),
        out_shape=jax.ShapeDtypeStruct((B, 1, KV), jnp.float32),
    )(query, weights,
      *[index_kv_cache] * NST,
      index_scratch.reshape(B, 1, KV))
    return out.reshape(B, KV)


def _topk_body(scores_hbm, lens_hbm, out_s_hbm, out_i_hbm,
               sc_v, key_a, key_b, idx_a, idx_b, lens_v,
               outs_v, outi_v, tots, bases, *hists):
    wid = lax.axis_index("s") * 2 + lax.axis_index("c")
    pltpu.sync_copy(scores_hbm.at[wid], sc_v)
    pltpu.sync_copy(lens_hbm.at[wid], lens_v)

    iota = lax.iota(jnp.int32, NLANE)
    len_vec = jnp.maximum(lens_v[...], jnp.int32(K))

    def pad(x):
        return x + lax.shift_right_logical(x, 9)

    @pl.loop(0, CHUNK, unroll=4)
    def _build(it):
        sl = pl.ds(it * NLANE, NLANE)
        bits = lax.bitcast_convert_type(sc_v[sl], jnp.int32)
        key_m = jnp.where(bits >= 0, bits ^ jnp.int32(-2**31), ~bits)
        invkey = ~key_m
        e = it * NLANE + iota
        invkey = jnp.where(e < len_vec, invkey, jnp.int32(-1))
        pa = pad(e)
        plsc.store_scatter(key_a, [pa], invkey)
        plsc.store_scatter(idx_a, [pa], e)

    ones = jnp.ones((NLANE,), jnp.int32)
    lane_base = iota * (CHUNK + 1)
    dmask = jnp.int32(RADIX - 1)

    for p in range(NPASS):
        shift = jnp.int32(DIGIT_BITS * p)
        src_k, src_i = (key_a, idx_a) if p % 2 == 0 else (key_b, idx_b)
        dst_k, dst_i = (key_b, idx_b) if p % 2 == 0 else (key_a, idx_a)

        @pl.loop(0, NBINS // NLANE, unroll=8)
        def _zero(i):
            sl = pl.ds(i * NLANE, NLANE)
            for hb in hists:
                hb[sl] = jnp.zeros((NLANE,), jnp.int32)

        @pl.loop(0, SUB)
        def _hist(it):
            ks = [plsc.load_gather(src_k, [lane_base + (s * SUB + it)])
                  for s in range(NB)]
            cidx = [(lax.shift_right_logical(k, shift) & dmask) * NLANE + iota
                    for k in ks]
            for s, hb in enumerate(hists):
                plsc.addupdate_scatter(hb, [cidx[s]], ones)

        @pl.loop(0, RADIX, unroll=2)
        def _local_prefix(dd):
            sl = pl.ds(dd * NLANE, NLANE)
            c = [hb[sl] for hb in hists]
            tot = c[0]
            for s in range(1, NB):
                tot = tot + c[s]
            incl = plsc.cumsum(tot)
            run = incl - tot
            plsc.store_scatter(tots, [jnp.full((NLANE,), dd, jnp.int32)],
                               incl, mask=iota == jnp.int32(NLANE - 1))
            for s, hb in enumerate(hists):
                hb[sl] = run
                if s + 1 < NB:
                    run = run + c[s]

        def _digit_prefix(g, carry):
            sl = pl.ds(g * NLANE, NLANE)
            tv = tots[sl]
            ex = plsc.cumsum(tv) - tv
            bases[sl] = ex + carry
            return carry + jnp.sum(tv)

        lax.fori_loop(0, RADIX // NLANE, _digit_prefix, jnp.int32(0))

        @pl.loop(0, SUB)
        def _permute(it):
            ks, vs = [], []
            for s in range(NB):
                a = lane_base + (s * SUB + it)
                ks.append(plsc.load_gather(src_k, [a]))
                vs.append(plsc.load_gather(src_i, [a]))
            ds = [lax.shift_right_logical(k, shift) & dmask for k in ks]
            cidx = [d * NLANE + iota for d in ds]
            pos = [plsc.load_gather(hists[s], [cidx[s]]) +
                   plsc.load_gather(bases, [ds[s]])
                   for s in range(NB)]
            pa = [pad(p) for p in pos]
            for s, hb in enumerate(hists):
                plsc.store_scatter(dst_k, [pa[s]], ks[s])
                plsc.store_scatter(dst_i, [pa[s]], vs[s])
                plsc.addupdate_scatter(hb, [cidx[s]], ones)

    src_i_fin = idx_a if NPASS % 2 == 0 else idx_b

    @pl.loop(0, K // NLANE, unroll=4)
    def _emit(it):
        sl = pl.ds(it * NLANE, NLANE)
        rank = it * NLANE + iota
        i_vec = plsc.load_gather(src_i_fin, [pad(rank)])
        outs_v[sl] = plsc.load_gather(sc_v, [i_vec])
        outi_v[sl] = i_vec

    pltpu.sync_copy(outs_v, out_s_hbm.at[wid])
    pltpu.sync_copy(outi_v, out_i_hbm.at[wid])


def _topk(scores, kv_lens):
    mesh = plsc.VectorSubcoreMesh(core_axis_name="c", subcore_axis_name="s")
    fn = pl.kernel(
        _topk_body,
        out_type=(jax.ShapeDtypeStruct((B, K), jnp.float32),
                  jax.ShapeDtypeStruct((B, K), jnp.int32)),
        mesh=mesh,
        scratch_types=[
            pltpu.VMEM((KV,), jnp.float32),
            pltpu.VMEM((KV + NLANE,), jnp.int32),
            pltpu.VMEM((KV + NLANE,), jnp.int32),
            pltpu.VMEM((KV + NLANE,), jnp.int32),
            pltpu.VMEM((KV + NLANE,), jnp.int32),
            pltpu.VMEM((NLANE,), jnp.int32),
            pltpu.VMEM((K,), jnp.float32),
            pltpu.VMEM((K,), jnp.int32),
            pltpu.VMEM((RADIX,), jnp.int32),
            pltpu.VMEM((RADIX,), jnp.int32),
        ] + [pltpu.VMEM((NBINS,), jnp.int32) for _ in range(NB)],
        compiler_params=pltpu.CompilerParams(needs_layout_passes=False),
    )
    return fn(scores, kv_lens)


def kernel(query, weights, index_kv_cache, kv_lens, block_size, layer_id,
           index_scratch):
    scores = _scores(query, weights, index_kv_cache, index_scratch)
    lens_b = jnp.broadcast_to(
        kv_lens.astype(jnp.int32)[:, None], (B, NLANE))
    return _topk(scores, lens_b)


# speedup vs baseline: 1.4359x; 1.0336x over previous
import functools

import jax
import jax.numpy as jnp
from jax import lax
from jax.experimental import pallas as pl
from jax.experimental.pallas import tpu as pltpu
from jax.experimental.pallas import tpu_sc as plsc

B, H, D = 32, 32, 128
KV = 8192
K = 2048
NLANE = 16
CHUNK = KV // NLANE
DIGIT_BITS = 7
RADIX = 1 << DIGIT_BITS
NPASS = 5
NB = 8
SUB = CHUNK // NB
NBINS = RADIX * NLANE

T_BLK = 4096
NST = 2


def _scores_body(q_ref, w_ref, *refs):
    kv_refs = refs[:NST]
    scr_ref, out_ref = refs[NST], refs[NST + 1]
    b = pl.program_id(0)
    t = pl.program_id(1)
    q = q_ref[0]
    w = w_ref[b][None, :]
    scale = jnp.float32(D) ** -0.5
    for j in range(NST):
        kv = kv_refs[j][0]
        logits = lax.dot_general(
            q, kv, (((1,), (1,)), ((), ())),
            preferred_element_type=jnp.float32)
        act = jnp.maximum(logits * scale, 0.0)
        s = lax.dot_general(
            w, act, (((1,), (0,)), ((), ())),
            preferred_element_type=jnp.float32)
        sl = pl.ds((t * NST + j) * T_BLK, T_BLK)
        out_ref[0, 0, sl] = s[0] + scr_ref[0, 0, sl]


def _scores(query, weights, index_kv_cache, index_scratch):
    grid = (B, KV // (NST * T_BLK))

    def kv_spec(j):
        return pl.BlockSpec((1, T_BLK, D),
                            lambda b, t, j=j: (b, t * NST + j, 0))

    out = pl.pallas_call(
        _scores_body,
        grid=grid,
        in_specs=[
            pl.BlockSpec((1, H, D), lambda b, t: (b, 0, 0)),
            pl.BlockSpec((B, H), lambda b, t: (0, 0)),
        ] + [kv_spec(j) for j in range(NST)] + [
            pl.BlockSpec((1, 1, KV), lambda b, t: (b, 0, 0)),
        ],
        out_specs=pl.BlockSpec((1, 1, KV), lambda b, t: (b, 0, 0)),
        out_shape=jax.ShapeDtypeStruct((B, 1, KV), jnp.float32),
    )(query, weights,
      *[index_kv_cache] * NST,
      index_scratch.reshape(B, 1, KV))
    return out.reshape(B, KV)


def _topk_body(scores_hbm, lens_hbm, out_s_hbm, out_i_hbm,
               sc_v, key_a, key_b, idx_a, idx_b, lens_v,
               outs_v, outi_v, tots, bases, *hists):
    wid = lax.axis_index("s") * 2 + lax.axis_index("c")
    pltpu.sync_copy(scores_hbm.at[wid], sc_v)
    pltpu.sync_copy(lens_hbm.at[wid], lens_v)

    iota = lax.iota(jnp.int32, NLANE)
    len_vec = jnp.maximum(lens_v[...], jnp.int32(K))

    def pad(x):
        return x + lax.shift_right_logical(x, 9)

    @pl.loop(0, CHUNK, unroll=4)
    def _build(it):
        sl = pl.ds(it * NLANE, NLANE)
        bits = lax.bitcast_convert_type(sc_v[sl], jnp.int32)
        key_m = jnp.where(bits >= 0, bits ^ jnp.int32(-2**31), ~bits)
        invkey = ~key_m
        e = it * NLANE + iota
        invkey = jnp.where(e < len_vec, invkey, jnp.int32(-1))
        pa = pad(e)
        plsc.store_scatter(key_a, [pa], invkey)
        plsc.store_scatter(idx_a, [pa], e)

    ones = jnp.ones((NLANE,), jnp.int32)
    lane_base = iota * (CHUNK + 1)
    dmask = jnp.int32(RADIX - 1)

    for p in range(NPASS):
        shift = jnp.int32(DIGIT_BITS * p)
        src_k, src_i = (key_a, idx_a) if p % 2 == 0 else (key_b, idx_b)
        dst_k, dst_i = (key_b, idx_b) if p % 2 == 0 else (key_a, idx_a)

        @pl.loop(0, NBINS // NLANE, unroll=8)
        def _zero(i):
            sl = pl.ds(i * NLANE, NLANE)
            for hb in hists:
                hb[sl] = jnp.zeros((NLANE,), jnp.int32)

        @pl.loop(0, SUB)
        def _hist(it):
            ks = [plsc.load_gather(src_k, [lane_base + (s * SUB + it)])
                  for s in range(NB)]
            cidx = [(lax.shift_right_logical(k, shift) & dmask) * NLANE + iota
                    for k in ks]
            for s, hb in enumerate(hists):
                plsc.addupdate_scatter(hb, [cidx[s]], ones)

        @pl.loop(0, RADIX, unroll=2)
        def _local_prefix(dd):
            sl = pl.ds(dd * NLANE, NLANE)
            c = [hb[sl] for hb in hists]
            tot = c[0]
            for s in range(1, NB):
                tot = tot + c[s]
            incl = plsc.cumsum(tot)
            run = incl - tot
            plsc.store_scatter(tots, [jnp.full((NLANE,), dd, jnp.int32)],
                               incl, mask=iota == jnp.int32(NLANE - 1))
            for s, hb in enumerate(hists):
                hb[sl] = run
                if s + 1 < NB:
                    run = run + c[s]

        def _digit_prefix(g, carry):
            sl = pl.ds(g * NLANE, NLANE)
            tv = tots[sl]
            ex = plsc.cumsum(tv) - tv
            bases[sl] = ex + carry
            return carry + jnp.sum(tv)

        lax.fori_loop(0, RADIX // NLANE, _digit_prefix, jnp.int32(0))

        @pl.loop(0, SUB)
        def _permute(it):
            ks, vs = [], []
            for s in range(NB):
                a = lane_base + (s * SUB + it)
                ks.append(plsc.load_gather(src_k, [a]))
                vs.append(plsc.load_gather(src_i, [a]))
            ds = [lax.shift_right_logical(k, shift) & dmask for k in ks]
            cidx = [d * NLANE + iota for d in ds]
            pos = [plsc.load_gather(hists[s], [cidx[s]]) +
                   plsc.load_gather(bases, [ds[s]])
                   for s in range(NB)]
            pa = [pad(p) for p in pos]
            for s, hb in enumerate(hists):
                plsc.store_scatter(dst_k, [pa[s]], ks[s])
                plsc.store_scatter(dst_i, [pa[s]], vs[s])
                plsc.addupdate_scatter(hb, [cidx[s]], ones)

    src_i_fin = idx_a if NPASS % 2 == 0 else idx_b

    @pl.loop(0, K // NLANE, unroll=4)
    def _emit(it):
        sl = pl.ds(it * NLANE, NLANE)
        rank = it * NLANE + iota
        i_vec = plsc.load_gather(src_i_fin, [pad(rank)])
        outs_v[sl] = plsc.load_gather(sc_v, [i_vec])
        outi_v[sl] = i_vec

    pltpu.sync_copy(outs_v, out_s_hbm.at[wid])
    pltpu.sync_copy(outi_v, out_i_hbm.at[wid])


def _topk(scores, kv_lens):
    mesh = plsc.VectorSubcoreMesh(core_axis_name="c", subcore_axis_name="s")
    fn = pl.kernel(
        _topk_body,
        out_type=(jax.ShapeDtypeStruct((B, K), jnp.float32),
                  jax.ShapeDtypeStruct((B, K), jnp.int32)),
        mesh=mesh,
        scratch_types=[
            pltpu.VMEM((KV,), jnp.float32),
            pltpu.VMEM((KV + NLANE,), jnp.int32),
            pltpu.VMEM((KV + NLANE,), jnp.int32),
            pltpu.VMEM((KV + NLANE,), jnp.int32),
            pltpu.VMEM((KV + NLANE,), jnp.int32),
            pltpu.VMEM((NLANE,), jnp.int32),
            pltpu.VMEM((K,), jnp.float32),
            pltpu.VMEM((K,), jnp.int32),
            pltpu.VMEM((RADIX,), jnp.int32),
            pltpu.VMEM((RADIX,), jnp.int32),
        ] + [pltpu.VMEM((NBINS,), jnp.int32) for _ in range(NB)],
        compiler_params=pltpu.CompilerParams(needs_layout_passes=False),
    )
    return fn(scores, kv_lens)


def kernel(query, weights, index_kv_cache, kv_lens, block_size, layer_id,
           index_scratch):
    scores = _scores(query, weights, index_kv_cache, index_scratch)
    lens_b = jnp.broadcast_to(
        kv_lens.astype(jnp.int32)[:, None], (B, NLANE))
    return _topk(scores, lens_b)


# speedup vs baseline: 1.4587x; 1.0158x over previous
import functools

import jax
import jax.numpy as jnp
from jax import lax
from jax.experimental import pallas as pl
from jax.experimental.pallas import tpu as pltpu
from jax.experimental.pallas import tpu_sc as plsc

B, H, D = 32, 32, 128
KV = 8192
K = 2048
NLANE = 16
CHUNK = KV // NLANE
DIGIT_BITS = 7
RADIX = 1 << DIGIT_BITS
NPASS = 5
NB = 8
SUB = CHUNK // NB
NBINS = RADIX * NLANE

T_BLK = 8192
NST = 1


def _scores_body(q_ref, w_ref, *refs):
    kv_refs = refs[:NST]
    scr_ref, out_ref = refs[NST], refs[NST + 1]
    b = pl.program_id(0)
    t = pl.program_id(1)
    q = q_ref[0]
    w = w_ref[b][None, :]
    scale = jnp.float32(D) ** -0.5
    for j in range(NST):
        kv = kv_refs[j][0]
        logits = lax.dot_general(
            q, kv, (((1,), (1,)), ((), ())),
            preferred_element_type=jnp.float32)
        act = jnp.maximum(logits * scale, 0.0)
        s = lax.dot_general(
            w, act, (((1,), (0,)), ((), ())),
            preferred_element_type=jnp.float32)
        sl = pl.ds((t * NST + j) * T_BLK, T_BLK)
        out_ref[0, 0, sl] = s[0] + scr_ref[0, 0, sl]


def _scores(query, weights, index_kv_cache, index_scratch):
    grid = (B, KV // (NST * T_BLK))

    def kv_spec(j):
        return pl.BlockSpec((1, T_BLK, D),
                            lambda b, t, j=j: (b, t * NST + j, 0))

    out = pl.pallas_call(
        _scores_body,
        grid=grid,
        in_specs=[
            pl.BlockSpec((1, H, D), lambda b, t: (b, 0, 0)),
            pl.BlockSpec((B, H), lambda b, t: (0, 0)),
        ] + [kv_spec(j) for j in range(NST)] + [
            pl.BlockSpec((1, 1, KV), lambda b, t: (b, 0, 0)),
        ],
        out_specs=pl.BlockSpec((1, 1, KV), lambda b, t: (b, 0, 0)),
        out_shape=jax.ShapeDtypeStruct((B, 1, KV), jnp.float32),
    )(query, weights,
      *[index_kv_cache] * NST,
      index_scratch.reshape(B, 1, KV))
    return out.reshape(B, KV)


def _topk_body(scores_hbm, lens_hbm, out_s_hbm, out_i_hbm,
               sc_v, key_a, key_b, idx_a, idx_b, lens_v,
               outs_v, outi_v, tots, bases, *hists):
    wid = lax.axis_index("s") * 2 + lax.axis_index("c")
    pltpu.sync_copy(scores_hbm.at[wid], sc_v)
    pltpu.sync_copy(lens_hbm.at[wid], lens_v)

    iota = lax.iota(jnp.int32, NLANE)
    len_vec = jnp.maximum(lens_v[...], jnp.int32(K))

    def pad(x):
        return x + lax.shift_right_logical(x, 9)

    @pl.loop(0, CHUNK, unroll=4)
    def _build(it):
        sl = pl.ds(it * NLANE, NLANE)
        bits = lax.bitcast_convert_type(sc_v[sl], jnp.int32)
        key_m = jnp.where(bits >= 0, bits ^ jnp.int32(-2**31), ~bits)
        invkey = ~key_m
        e = it * NLANE + iota
        invkey = jnp.where(e < len_vec, invkey, jnp.int32(-1))
        pa = pad(e)
        plsc.store_scatter(key_a, [pa], invkey)
        plsc.store_scatter(idx_a, [pa], e)

    ones = jnp.ones((NLANE,), jnp.int32)
    lane_base = iota * (CHUNK + 1)
    dmask = jnp.int32(RADIX - 1)

    for p in range(NPASS):
        shift = jnp.int32(DIGIT_BITS * p)
        src_k, src_i = (key_a, idx_a) if p % 2 == 0 else (key_b, idx_b)
        dst_k, dst_i = (key_b, idx_b) if p % 2 == 0 else (key_a, idx_a)

        @pl.loop(0, NBINS // NLANE, unroll=8)
        def _zero(i):
            sl = pl.ds(i * NLANE, NLANE)
            for hb in hists:
                hb[sl] = jnp.zeros((NLANE,), jnp.int32)

        @pl.loop(0, SUB)
        def _hist(it):
            ks = [plsc.load_gather(src_k, [lane_base + (s * SUB + it)])
                  for s in range(NB)]
            cidx = [(lax.shift_right_logical(k, shift) & dmask) * NLANE + iota
                    for k in ks]
            for s, hb in enumerate(hists):
                plsc.addupdate_scatter(hb, [cidx[s]], ones)

        @pl.loop(0, RADIX, unroll=2)
        def _local_prefix(dd):
            sl = pl.ds(dd * NLANE, NLANE)
            c = [hb[sl] for hb in hists]
            tot = c[0]
            for s in range(1, NB):
                tot = tot + c[s]
            incl = plsc.cumsum(tot)
            run = incl - tot
            plsc.store_scatter(tots, [jnp.full((NLANE,), dd, jnp.int32)],
                               incl, mask=iota == jnp.int32(NLANE - 1))
            for s, hb in enumerate(hists):
                hb[sl] = run
                if s + 1 < NB:
                    run = run + c[s]

        def _digit_prefix(g, carry):
            sl = pl.ds(g * NLANE, NLANE)
            tv = tots[sl]
            ex = plsc.cumsum(tv) - tv
            bases[sl] = ex + carry
            return carry + jnp.sum(tv)

        lax.fori_loop(0, RADIX // NLANE, _digit_prefix, jnp.int32(0))

        @pl.loop(0, SUB)
        def _permute(it):
            ks, vs = [], []
            for s in range(NB):
                a = lane_base + (s * SUB + it)
                ks.append(plsc.load_gather(src_k, [a]))
                vs.append(plsc.load_gather(src_i, [a]))
            ds = [lax.shift_right_logical(k, shift) & dmask for k in ks]
            cidx = [d * NLANE + iota for d in ds]
            pos = [plsc.load_gather(hists[s], [cidx[s]]) +
                   plsc.load_gather(bases, [ds[s]])
                   for s in range(NB)]
            pa = [pad(p) for p in pos]
            for s, hb in enumerate(hists):
                plsc.store_scatter(dst_k, [pa[s]], ks[s])
                plsc.store_scatter(dst_i, [pa[s]], vs[s])
                plsc.addupdate_scatter(hb, [cidx[s]], ones)

    src_i_fin = idx_a if NPASS % 2 == 0 else idx_b

    @pl.loop(0, K // NLANE, unroll=4)
    def _emit(it):
        sl = pl.ds(it * NLANE, NLANE)
        rank = it * NLANE + iota
        i_vec = plsc.load_gather(src_i_fin, [pad(rank)])
        outs_v[sl] = plsc.load_gather(sc_v, [i_vec])
        outi_v[sl] = i_vec

    pltpu.sync_copy(outs_v, out_s_hbm.at[wid])
    pltpu.sync_copy(outi_v, out_i_hbm.at[wid])


def _topk(scores, kv_lens):
    mesh = plsc.VectorSubcoreMesh(core_axis_name="c", subcore_axis_name="s")
    fn = pl.kernel(
        _topk_body,
        out_type=(jax.ShapeDtypeStruct((B, K), jnp.float32),
                  jax.ShapeDtypeStruct((B, K), jnp.int32)),
        mesh=mesh,
        scratch_types=[
            pltpu.VMEM((KV,), jnp.float32),
            pltpu.VMEM((KV + NLANE,), jnp.int32),
            pltpu.VMEM((KV + NLANE,), jnp.int32),
            pltpu.VMEM((KV + NLANE,), jnp.int32),
            pltpu.VMEM((KV + NLANE,), jnp.int32),
            pltpu.VMEM((NLANE,), jnp.int32),
            pltpu.VMEM((K,), jnp.float32),
            pltpu.VMEM((K,), jnp.int32),
            pltpu.VMEM((RADIX,), jnp.int32),
            pltpu.VMEM((RADIX,), jnp.int32),
        ] + [pltpu.VMEM((NBINS,), jnp.int32) for _ in range(NB)],
        compiler_params=pltpu.CompilerParams(needs_layout_passes=False),
    )
    return fn(scores, kv_lens)


def kernel(query, weights, index_kv_cache, kv_lens, block_size, layer_id,
           index_scratch):
    scores = _scores(query, weights, index_kv_cache, index_scratch)
    lens_b = jnp.broadcast_to(
        kv_lens.astype(jnp.int32)[:, None], (B, NLANE))
    return _topk(scores, lens_b)


# speedup vs baseline: 1.5510x; 1.0633x over previous
import functools

import jax
import jax.numpy as jnp
from jax import lax
from jax.experimental import pallas as pl
from jax.experimental.pallas import tpu as pltpu
from jax.experimental.pallas import tpu_sc as plsc

B, H, D = 32, 32, 128
KV = 8192
K = 2048
NLANE = 16
CHUNK = KV // NLANE
DIGIT_BITS = 7
RADIX = 1 << DIGIT_BITS
NPASS = 5
NB = 8
SUB = CHUNK // NB
NBINS = RADIX * NLANE

T_BLK = 8192
NST = 1


def _scores_body(q_ref, w_ref, kv_ref, out_ref):
    b = pl.program_id(0)
    q = q_ref[0]
    w = w_ref[b][None, :]
    scale = jnp.float32(D) ** -0.5
    kv = kv_ref[0]
    logits = lax.dot_general(
        q, kv, (((1,), (1,)), ((), ())),
        preferred_element_type=jnp.float32)
    act = jnp.maximum(logits * scale, 0.0)
    s = lax.dot_general(
        w, act, (((1,), (0,)), ((), ())),
        preferred_element_type=jnp.float32)
    out_ref[pl.ds(b % 8, 1), :] = s


def _scores(query, weights, index_kv_cache):
    return pl.pallas_call(
        _scores_body,
        grid=(B,),
        in_specs=[
            pl.BlockSpec((1, H, D), lambda b: (b, 0, 0)),
            pl.BlockSpec((B, H), lambda b: (0, 0)),
            pl.BlockSpec((1, KV, D), lambda b: (b, 0, 0)),
        ],
        out_specs=pl.BlockSpec((8, KV), lambda b: (b // 8, 0)),
        out_shape=jax.ShapeDtypeStruct((B, KV), jnp.float32),
    )(query, weights, index_kv_cache)


def _topk_body(scores_hbm, scr_hbm, lens_hbm, out_s_hbm, out_i_hbm,
               sc_v, scr_v, key_a, key_b, idx_a, idx_b, lens_v,
               outs_v, outi_v, tots, bases, *hists):
    wid = lax.axis_index("s") * 2 + lax.axis_index("c")
    pltpu.sync_copy(scores_hbm.at[wid], sc_v)
    pltpu.sync_copy(scr_hbm.at[wid], scr_v)
    pltpu.sync_copy(lens_hbm.at[wid], lens_v)

    iota = lax.iota(jnp.int32, NLANE)
    len_vec = jnp.maximum(lens_v[...], jnp.int32(K))

    def pad(x):
        return x + lax.shift_right_logical(x, 9)

    BU = 4

    @pl.loop(0, CHUNK // BU)
    def _build(bt):
        sls = [pl.ds((bt * BU + u) * NLANE, NLANE) for u in range(BU)]
        ss = [sc_v[sl] + scr_v[sl] for sl in sls]
        kvs, pas = [], []
        for u in range(BU):
            bits = lax.bitcast_convert_type(ss[u], jnp.int32)
            key_m = jnp.where(bits >= 0, bits ^ jnp.int32(-2**31), ~bits)
            invkey = ~key_m
            e = (bt * BU + u) * NLANE + iota
            invkey = jnp.where(e < len_vec, invkey, jnp.int32(-1))
            kvs.append((invkey, e))
            pas.append(pad(e))
        for u in range(BU):
            sc_v[sls[u]] = ss[u]
            plsc.store_scatter(key_a, [pas[u]], kvs[u][0])
            plsc.store_scatter(idx_a, [pas[u]], kvs[u][1])

    ones = jnp.ones((NLANE,), jnp.int32)
    lane_base = iota * (CHUNK + 1)
    dmask = jnp.int32(RADIX - 1)

    for p in range(NPASS):
        shift = jnp.int32(DIGIT_BITS * p)
        src_k, src_i = (key_a, idx_a) if p % 2 == 0 else (key_b, idx_b)
        dst_k, dst_i = (key_b, idx_b) if p % 2 == 0 else (key_a, idx_a)

        @pl.loop(0, NBINS // NLANE, unroll=8)
        def _zero(i):
            sl = pl.ds(i * NLANE, NLANE)
            for hb in hists:
                hb[sl] = jnp.zeros((NLANE,), jnp.int32)

        @pl.loop(0, SUB)
        def _hist(it):
            ks = [plsc.load_gather(src_k, [lane_base + (s * SUB + it)])
                  for s in range(NB)]
            cidx = [(lax.shift_right_logical(k, shift) & dmask) * NLANE + iota
                    for k in ks]
            for s, hb in enumerate(hists):
                plsc.addupdate_scatter(hb, [cidx[s]], ones)

        @pl.loop(0, RADIX, unroll=2)
        def _local_prefix(dd):
            sl = pl.ds(dd * NLANE, NLANE)
            c = [hb[sl] for hb in hists]
            tot = c[0]
            for s in range(1, NB):
                tot = tot + c[s]
            incl = plsc.cumsum(tot)
            run = incl - tot
            plsc.store_scatter(tots, [jnp.full((NLANE,), dd, jnp.int32)],
                               incl, mask=iota == jnp.int32(NLANE - 1))
            for s, hb in enumerate(hists):
                hb[sl] = run
                if s + 1 < NB:
                    run = run + c[s]

        def _digit_prefix(g, carry):
            sl = pl.ds(g * NLANE, NLANE)
            tv = tots[sl]
            ex = plsc.cumsum(tv) - tv
            bases[sl] = ex + carry
            return carry + jnp.sum(tv)

        lax.fori_loop(0, RADIX // NLANE, _digit_prefix, jnp.int32(0))

        @pl.loop(0, SUB)
        def _permute(it):
            ks, vs = [], []
            for s in range(NB):
                a = lane_base + (s * SUB + it)
                ks.append(plsc.load_gather(src_k, [a]))
                vs.append(plsc.load_gather(src_i, [a]))
            ds = [lax.shift_right_logical(k, shift) & dmask for k in ks]
            cidx = [d * NLANE + iota for d in ds]
            pos = [plsc.load_gather(hists[s], [cidx[s]]) +
                   plsc.load_gather(bases, [ds[s]])
                   for s in range(NB)]
            pa = [pad(p) for p in pos]
            for s, hb in enumerate(hists):
                plsc.store_scatter(dst_k, [pa[s]], ks[s])
                plsc.store_scatter(dst_i, [pa[s]], vs[s])
                plsc.addupdate_scatter(hb, [cidx[s]], ones)

    src_i_fin = idx_a if NPASS % 2 == 0 else idx_b

    @pl.loop(0, K // NLANE, unroll=4)
    def _emit(it):
        sl = pl.ds(it * NLANE, NLANE)
        rank = it * NLANE + iota
        i_vec = plsc.load_gather(src_i_fin, [pad(rank)])
        outs_v[sl] = plsc.load_gather(sc_v, [i_vec])
        outi_v[sl] = i_vec

    pltpu.sync_copy(outs_v, out_s_hbm.at[wid])
    pltpu.sync_copy(outi_v, out_i_hbm.at[wid])


def _topk(scores, index_scratch, kv_lens):
    mesh = plsc.VectorSubcoreMesh(core_axis_name="c", subcore_axis_name="s")
    fn = pl.kernel(
        _topk_body,
        out_type=(jax.ShapeDtypeStruct((B, K), jnp.float32),
                  jax.ShapeDtypeStruct((B, K), jnp.int32)),
        mesh=mesh,
        scratch_types=[
            pltpu.VMEM((KV,), jnp.float32),
            pltpu.VMEM((KV,), jnp.float32),
            pltpu.VMEM((KV + NLANE,), jnp.int32),
            pltpu.VMEM((KV + NLANE,), jnp.int32),
            pltpu.VMEM((KV + NLANE,), jnp.int32),
            pltpu.VMEM((KV + NLANE,), jnp.int32),
            pltpu.VMEM((NLANE,), jnp.int32),
            pltpu.VMEM((K,), jnp.float32),
            pltpu.VMEM((K,), jnp.int32),
            pltpu.VMEM((RADIX,), jnp.int32),
            pltpu.VMEM((RADIX,), jnp.int32),
        ] + [pltpu.VMEM((NBINS,), jnp.int32) for _ in range(NB)],
        compiler_params=pltpu.CompilerParams(needs_layout_passes=False),
    )
    return fn(scores, index_scratch, kv_lens)


def kernel(query, weights, index_kv_cache, kv_lens, block_size, layer_id,
           index_scratch):
    scores = _scores(query, weights, index_kv_cache)
    lens_b = jnp.broadcast_to(
        kv_lens.astype(jnp.int32)[:, None], (B, NLANE))
    return _topk(scores, index_scratch, lens_b)


# speedup vs baseline: 1.5748x; 1.0153x over previous
import functools

import jax
import jax.numpy as jnp
from jax import lax
from jax.experimental import pallas as pl
from jax.experimental.pallas import tpu as pltpu
from jax.experimental.pallas import tpu_sc as plsc

B, H, D = 32, 32, 128
KV = 8192
K = 2048
NLANE = 16
CHUNK = KV // NLANE
DIGIT_BITS = 7
RADIX = 1 << DIGIT_BITS
NPASS = 5
NB = 8
SUB = CHUNK // NB
NBINS = RADIX * NLANE

T_BLK = 8192
NST = 1


def _scores_body(q_ref, w_ref, kv_ref, out_ref):
    b = pl.program_id(0)
    q = q_ref[0]
    w = w_ref[b][None, :]
    scale = jnp.float32(D) ** -0.5
    kv = kv_ref[0]
    logits = lax.dot_general(
        q, kv, (((1,), (1,)), ((), ())),
        preferred_element_type=jnp.float32)
    act = jnp.maximum(logits * scale, 0.0)
    s = jnp.sum(act * w_ref[b][:, None], axis=0, keepdims=True)
    out_ref[pl.ds(b % 8, 1), :] = s


def _scores(query, weights, index_kv_cache):
    return pl.pallas_call(
        _scores_body,
        grid=(B,),
        in_specs=[
            pl.BlockSpec((1, H, D), lambda b: (b, 0, 0)),
            pl.BlockSpec((B, H), lambda b: (0, 0)),
            pl.BlockSpec((1, KV, D), lambda b: (b, 0, 0)),
        ],
        out_specs=pl.BlockSpec((8, KV), lambda b: (b // 8, 0)),
        out_shape=jax.ShapeDtypeStruct((B, KV), jnp.float32),
    )(query, weights, index_kv_cache)


def _topk_body(scores_hbm, scr_hbm, lens_hbm, out_s_hbm, out_i_hbm,
               sc_v, scr_v, key_a, key_b, idx_a, idx_b, lens_v,
               outs_v, outi_v, tots, bases, *hists):
    wid = lax.axis_index("s") * 2 + lax.axis_index("c")
    pltpu.sync_copy(scores_hbm.at[wid], sc_v)
    pltpu.sync_copy(scr_hbm.at[wid], scr_v)
    pltpu.sync_copy(lens_hbm.at[wid], lens_v)

    iota = lax.iota(jnp.int32, NLANE)
    len_vec = jnp.maximum(lens_v[...], jnp.int32(K))

    def pad(x):
        return x + lax.shift_right_logical(x, 9)

    BU = 4

    @pl.loop(0, CHUNK // BU)
    def _build(bt):
        sls = [pl.ds((bt * BU + u) * NLANE, NLANE) for u in range(BU)]
        ss = [sc_v[sl] + scr_v[sl] for sl in sls]
        kvs, pas = [], []
        for u in range(BU):
            bits = lax.bitcast_convert_type(ss[u], jnp.int32)
            key_m = jnp.where(bits >= 0, bits ^ jnp.int32(-2**31), ~bits)
            invkey = ~key_m
            e = (bt * BU + u) * NLANE + iota
            invkey = jnp.where(e < len_vec, invkey, jnp.int32(-1))
            kvs.append((invkey, e))
            pas.append(pad(e))
        for u in range(BU):
            sc_v[sls[u]] = ss[u]
            plsc.store_scatter(key_a, [pas[u]], kvs[u][0])
            plsc.store_scatter(idx_a, [pas[u]], kvs[u][1])

    ones = jnp.ones((NLANE,), jnp.int32)
    lane_base = iota * (CHUNK + 1)
    dmask = jnp.int32(RADIX - 1)

    for p in range(NPASS):
        shift = jnp.int32(DIGIT_BITS * p)
        src_k, src_i = (key_a, idx_a) if p % 2 == 0 else (key_b, idx_b)
        dst_k, dst_i = (key_b, idx_b) if p % 2 == 0 else (key_a, idx_a)

        @pl.loop(0, NBINS // NLANE, unroll=8)
        def _zero(i):
            sl = pl.ds(i * NLANE, NLANE)
            for hb in hists:
                hb[sl] = jnp.zeros((NLANE,), jnp.int32)

        @pl.loop(0, SUB)
        def _hist(it):
            ks = [plsc.load_gather(src_k, [lane_base + (s * SUB + it)])
                  for s in range(NB)]
            cidx = [(lax.shift_right_logical(k, shift) & dmask) * NLANE + iota
                    for k in ks]
            for s, hb in enumerate(hists):
                plsc.addupdate_scatter(hb, [cidx[s]], ones)

        @pl.loop(0, RADIX, unroll=2)
        def _local_prefix(dd):
            sl = pl.ds(dd * NLANE, NLANE)
            c = [hb[sl] for hb in hists]
            tot = c[0]
            for s in range(1, NB):
                tot = tot + c[s]
            incl = plsc.cumsum(tot)
            run = incl - tot
            plsc.store_scatter(tots, [jnp.full((NLANE,), dd, jnp.int32)],
                               incl, mask=iota == jnp.int32(NLANE - 1))
            for s, hb in enumerate(hists):
                hb[sl] = run
                if s + 1 < NB:
                    run = run + c[s]

        def _digit_prefix(g, carry):
            sl = pl.ds(g * NLANE, NLANE)
            tv = tots[sl]
            ex = plsc.cumsum(tv) - tv
            bases[sl] = ex + carry
            return carry + jnp.sum(tv)

        lax.fori_loop(0, RADIX // NLANE, _digit_prefix, jnp.int32(0))

        @pl.loop(0, SUB)
        def _permute(it):
            ks, vs = [], []
            for s in range(NB):
                a = lane_base + (s * SUB + it)
                ks.append(plsc.load_gather(src_k, [a]))
                vs.append(plsc.load_gather(src_i, [a]))
            ds = [lax.shift_right_logical(k, shift) & dmask for k in ks]
            cidx = [d * NLANE + iota for d in ds]
            pos = [plsc.load_gather(hists[s], [cidx[s]]) +
                   plsc.load_gather(bases, [ds[s]])
                   for s in range(NB)]
            pa = [pad(p) for p in pos]
            for s, hb in enumerate(hists):
                plsc.store_scatter(dst_k, [pa[s]], ks[s])
                plsc.store_scatter(dst_i, [pa[s]], vs[s])
                plsc.addupdate_scatter(hb, [cidx[s]], ones)

    src_i_fin = idx_a if NPASS % 2 == 0 else idx_b

    @pl.loop(0, K // NLANE, unroll=4)
    def _emit(it):
        sl = pl.ds(it * NLANE, NLANE)
        rank = it * NLANE + iota
        i_vec = plsc.load_gather(src_i_fin, [pad(rank)])
        outs_v[sl] = plsc.load_gather(sc_v, [i_vec])
        outi_v[sl] = i_vec

    pltpu.sync_copy(outs_v, out_s_hbm.at[wid])
    pltpu.sync_copy(outi_v, out_i_hbm.at[wid])


def _topk(scores, index_scratch, kv_lens):
    mesh = plsc.VectorSubcoreMesh(core_axis_name="c", subcore_axis_name="s")
    fn = pl.kernel(
        _topk_body,
        out_type=(jax.ShapeDtypeStruct((B, K), jnp.float32),
                  jax.ShapeDtypeStruct((B, K), jnp.int32)),
        mesh=mesh,
        scratch_types=[
            pltpu.VMEM((KV,), jnp.float32),
            pltpu.VMEM((KV,), jnp.float32),
            pltpu.VMEM((KV + NLANE,), jnp.int32),
            pltpu.VMEM((KV + NLANE,), jnp.int32),
            pltpu.VMEM((KV + NLANE,), jnp.int32),
            pltpu.VMEM((KV + NLANE,), jnp.int32),
            pltpu.VMEM((NLANE,), jnp.int32),
            pltpu.VMEM((K,), jnp.float32),
            pltpu.VMEM((K,), jnp.int32),
            pltpu.VMEM((RADIX,), jnp.int32),
            pltpu.VMEM((RADIX,), jnp.int32),
        ] + [pltpu.VMEM((NBINS,), jnp.int32) for _ in range(NB)],
        compiler_params=pltpu.CompilerParams(needs_layout_passes=False),
    )
    return fn(scores, index_scratch, kv_lens)


def kernel(query, weights, index_kv_cache, kv_lens, block_size, layer_id,
           index_scratch):
    scores = _scores(query, weights, index_kv_cache)
    lens_b = jnp.broadcast_to(
        kv_lens.astype(jnp.int32)[:, None], (B, NLANE))
    return _topk(scores, index_scratch, lens_b)
